# baseline (speedup 1.0000x reference)
import functools

import jax
import jax.numpy as jnp
from jax import lax
from jax.experimental import pallas as pl
from jax.experimental.pallas import tpu as pltpu
from jax.experimental.pallas import tpu_sc as plsc

N = 10000
E = 320000
D_IN = 128
D_HID = 128
D_OUT = 64
DW = 128

NC = 2
NS = 16
NT = NC * NS
NPAD = 10240
SLAB = NPAD // NS
CH = 80
NCH = (E // NT) // CH

_MESH = plsc.VectorSubcoreMesh(
    core_axis_name="c", subcore_axis_name="s", num_cores=NC, num_subcores=NS
)


@functools.partial(
    pl.kernel,
    out_type=jax.ShapeDtypeStruct((NC * NPAD,), jnp.float32),
    mesh=_MESH,
    scratch_types=[
        pltpu.VMEM((NCH, CH), jnp.int32),
        pltpu.VMEM((CH,), jnp.float32),
        pltpu.VMEM_SHARED((NPAD,), jnp.float32),
    ],
)
def _deg_kernel(dst3d, zeros1, deg_out, idx_d, ones_v, sdeg):
    c = lax.axis_index("c")
    s = lax.axis_index("s")
    t = c * NS + s
    pltpu.sync_copy(zeros1.at[pl.ds(s * SLAB, SLAB)], sdeg.at[pl.ds(s * SLAB, SLAB)])
    pltpu.sync_copy(dst3d.at[t], idx_d)
    ones16 = jnp.full((16,), 1.0, dtype=jnp.float32)
    for j in range(CH // 16):
        ones_v[pl.ds(j * 16, 16)] = ones16
    plsc.subcore_barrier()

    def body(i, carry):
        pltpu.sync_copy(ones_v, sdeg.at[idx_d.at[i]], add=True)
        return carry

    lax.fori_loop(0, NCH, body, 0)
    plsc.subcore_barrier()
    pltpu.sync_copy(
        sdeg.at[pl.ds(s * SLAB, SLAB)],
        deg_out.at[pl.ds(c * NPAD + s * SLAB, SLAB)],
    )


def _make_scatter(ds, nr, ni, slag, gla, ila, tc_tiling=True):
    assert ni >= ila + slag + 1 and nr >= gla + slag

    @functools.partial(
        pl.kernel,
        out_type=jax.ShapeDtypeStruct((NC * NPAD, ds), jnp.float32),
        mesh=_MESH,
        compiler_params=pltpu.CompilerParams(use_tc_tiling_on_sc=tc_tiling),
        scratch_types=[
            pltpu.VMEM((ni, CH), jnp.int32),
            pltpu.VMEM((ni, CH), jnp.int32),
            pltpu.VMEM((nr, CH, ds), jnp.float32),
            pltpu.SemaphoreType.DMA,
            pltpu.SemaphoreType.DMA,
            pltpu.SemaphoreType.DMA,
            pltpu.VMEM_SHARED((NPAD, ds), jnp.float32),
        ],
    )
    def _scatter(h, src3d, dst3d, zeros2, acc_out,
                 idx_s, idx_d, rows, gsem, isem, ssem, sacc):
        c = lax.axis_index("c")
        s = lax.axis_index("s")
        t = c * NS + s
        slab = pl.ds(s * SLAB, SLAB)
        pltpu.sync_copy(zeros2.at[slab], sacc.at[slab])
        plsc.subcore_barrier()

        def load_idx(j, slot):
            pltpu.async_copy(src3d.at[t, j], idx_s.at[slot], isem)
            pltpu.async_copy(dst3d.at[t, j], idx_d.at[slot], isem)

        def wait_idx(slot):
            pltpu.make_async_copy(src3d.at[t, 0], idx_s.at[slot], isem).wait()
            pltpu.make_async_copy(dst3d.at[t, 0], idx_d.at[slot], isem).wait()

        def gather(slot_i, slot_r):
            pltpu.async_copy(h.at[idx_s.at[slot_i]], rows.at[slot_r], gsem)

        def wait_gather(slot_i, slot_r):
            pltpu.make_async_copy(
                h.at[idx_s.at[slot_i]], rows.at[slot_r], gsem
            ).wait()

        def scat(slot_i, slot_r):
            pltpu.async_copy(
                rows.at[slot_r], sacc.at[idx_d.at[slot_i]], ssem, add=True
            )

        def wait_scat(slot_i, slot_r):
            pltpu.make_async_copy(
                rows.at[slot_r], sacc.at[idx_d.at[slot_i]], ssem
            ).wait()

        for j in range(ila):
            load_idx(j, j)
        for j in range(gla):
            wait_idx(j)
            gather(j, j)

        def body(i, carry):
            ri = lax.rem(i, nr)
            ii = lax.rem(i, ni)
            wait_gather(ii, ri)
            scat(ii, ri)

            @pl.when(i - slag >= 0)
            def _():
                wait_scat(lax.rem(i - slag, ni), lax.rem(i - slag, nr))

            @pl.when(i + gla < NCH)
            def _():
                wait_idx(lax.rem(i + gla, ni))
                gather(lax.rem(i + gla, ni), lax.rem(i + gla, nr))

            @pl.when(i + ila < NCH)
            def _():
                load_idx(i + ila, lax.rem(i + ila, ni))

            return carry

        lax.fori_loop(0, NCH, body, 0)
        for j in range(slag, 0, -1):
            wait_scat(lax.rem(NCH - j, ni), lax.rem(NCH - j, nr))
        plsc.subcore_barrier()
        pltpu.sync_copy(
            sacc.at[slab], acc_out.at[pl.ds(c * NPAD + s * SLAB, SLAB)]
        )

    return _scatter


_scatter_l1 = _make_scatter(ds=DW, nr=4, ni=6, slag=2, gla=2, ila=3)
_scatter_l2 = _make_scatter(ds=D_OUT, nr=6, ni=10, slag=3, gla=3, ila=6,
                            tc_tiling=False)


def _k1a_body(x_ref, w_ref, o_ref):
    o_ref[...] = jnp.dot(x_ref[...], w_ref[...],
                         preferred_element_type=jnp.float32)


def _k1b_body(h_ref, deg2_ref, o_ref, dinv_ref):
    d2 = deg2_ref[...]
    d = d2[:, 0:1] + d2[:, 1:2] + 1.0
    dinv = lax.rsqrt(d)
    dinv_ref[...] = dinv
    o_ref[...] = dinv * h_ref[...]


def _k2_body(acc_ref, h1p_ref, dinv_ref, b1_ref, w2_ref, o_ref):
    dinv = dinv_ref[...]
    t = dinv * (acc_ref[0] + acc_ref[1] + h1p_ref[...]) + b1_ref[...]
    r = jnp.maximum(t, 0.0)
    h2 = jnp.dot(r, w2_ref[...], preferred_element_type=jnp.float32)
    o_ref[...] = dinv * h2


def _k3_body(acc_ref, h2p_ref, dinv_ref, b2_ref, o_ref):
    acc = acc_ref[0] + acc_ref[1] + h2p_ref[...]
    t = dinv_ref[...] * acc + b2_ref[...]
    m = jnp.max(t, axis=1, keepdims=True)
    e = jnp.exp(t - m)
    ssum = jnp.sum(e, axis=1, keepdims=True)
    o_ref[...] = (t - m) - jnp.log(ssum)


def _full(shape):
    return pl.BlockSpec(shape, lambda i: tuple(0 for _ in shape))


def kernel(x, edge_index, W1, b1, W2, b2):
    ei = edge_index.astype(jnp.int32)
    src3d = ei[0].reshape(NT, NCH, CH)
    dst3d = ei[1].reshape(NT, NCH, CH)
    zeros1 = jnp.zeros((NPAD,), jnp.float32)
    zeros2 = jnp.zeros((NPAD, DW), jnp.float32)
    zeros64 = jnp.zeros((NPAD, D_OUT), jnp.float32)

    h1 = pl.pallas_call(
        _k1a_body,
        out_shape=jax.ShapeDtypeStruct((N, D_HID), jnp.float32),
    )(x, W1)

    degflat = _deg_kernel(dst3d, zeros1)
    deg2 = degflat.reshape(NC, NPAD).T[:N]

    h1p, dinv = pl.pallas_call(
        _k1b_body,
        out_shape=[
            jax.ShapeDtypeStruct((N, D_HID), jnp.float32),
            jax.ShapeDtypeStruct((N, 1), jnp.float32),
        ],
    )(h1, deg2)

    acc1 = _scatter_l1(h1p, src3d, dst3d, zeros2).reshape(NC, NPAD, DW)

    h2p = pl.pallas_call(
        _k2_body,
        grid=(1,),
        in_specs=[
            _full((NC, N, DW)),
            _full((N, D_HID)),
            _full((N, 1)),
            _full((1, D_HID)),
            _full((D_HID, D_OUT)),
        ],
        out_specs=_full((N, D_OUT)),
        out_shape=jax.ShapeDtypeStruct((N, D_OUT), jnp.float32),
    )(acc1, h1p, dinv, b1.reshape(1, D_HID), W2)

    acc2 = _scatter_l2(h2p, src3d, dst3d, zeros64).reshape(NC, NPAD, D_OUT)

    out = pl.pallas_call(
        _k3_body,
        grid=(1,),
        in_specs=[
            _full((NC, N, D_OUT)),
            _full((N, D_OUT)),
            _full((N, 1)),
            _full((1, D_OUT)),
        ],
        out_specs=_full((N, D_OUT)),
        out_shape=jax.ShapeDtypeStruct((N, D_OUT), jnp.float32),
    )(acc2, h2p, dinv, b2.reshape(1, D_OUT))
    return out


# speedup vs baseline: 1.0287x; 1.0287x over previous
import functools

import jax
import jax.numpy as jnp
from jax import lax
from jax.experimental import pallas as pl
from jax.experimental.pallas import tpu as pltpu
from jax.experimental.pallas import tpu_sc as plsc

N = 10000
E = 320000
D_IN = 128
D_HID = 128
D_OUT = 64
DW = 128

NC = 2
NS = 16
NT = NC * NS
NPAD = 10240
SLAB = NPAD // NS
CH = 80
NCH = (E // NT) // CH

_MESH = plsc.VectorSubcoreMesh(
    core_axis_name="c", subcore_axis_name="s", num_cores=NC, num_subcores=NS
)


ET = E // NT
DNI = 8
DILA = 4
DSLAG = 3


@functools.partial(
    pl.kernel,
    out_type=jax.ShapeDtypeStruct((NC * NPAD,), jnp.float32),
    mesh=_MESH,
    scratch_types=[
        pltpu.VMEM((DNI, CH), jnp.int32),
        pltpu.VMEM((CH,), jnp.float32),
        pltpu.VMEM((SLAB,), jnp.float32),
        pltpu.SemaphoreType.DMA,
        pltpu.SemaphoreType.DMA,
        pltpu.VMEM_SHARED((NPAD,), jnp.float32),
    ],
)
def _deg_kernel(dst1d, deg_out, idx_d, ones_v, zbuf, isem, ssem, sdeg):
    c = lax.axis_index("c")
    s = lax.axis_index("s")
    t = c * NS + s
    slab = pl.ds(s * SLAB, SLAB)
    z16 = jnp.zeros((16,), dtype=jnp.float32)
    ones16 = jnp.full((16,), 1.0, dtype=jnp.float32)

    def fill(i, carry):
        zbuf[pl.ds(i * 16, 16)] = z16
        return carry

    lax.fori_loop(0, SLAB // 16, fill, 0)
    for j in range(CH // 16):
        ones_v[pl.ds(j * 16, 16)] = ones16
    pltpu.sync_copy(zbuf, sdeg.at[slab])

    def load_idx(j, slot):
        base = pl.multiple_of(t * ET + j * CH, CH)
        pltpu.async_copy(dst1d.at[pl.ds(base, CH)], idx_d.at[slot], isem)

    def wait_idx(slot):
        pltpu.make_async_copy(dst1d.at[pl.ds(0, CH)], idx_d.at[slot], isem).wait()

    for j in range(DILA):
        load_idx(j, j)
    plsc.subcore_barrier()

    def body(i, carry):
        ii = lax.rem(i, DNI)
        wait_idx(ii)
        pltpu.async_copy(ones_v, sdeg.at[idx_d.at[ii]], ssem, add=True)

        @pl.when(i - DSLAG >= 0)
        def _():
            io = lax.rem(i - DSLAG, DNI)
            pltpu.make_async_copy(ones_v, sdeg.at[idx_d.at[io]], ssem).wait()

        @pl.when(i + DILA < NCH)
        def _():
            load_idx(i + DILA, lax.rem(i + DILA, DNI))

        return carry

    lax.fori_loop(0, NCH, body, 0)
    for j in range(DSLAG, 0, -1):
        io = lax.rem(NCH - j, DNI)
        pltpu.make_async_copy(ones_v, sdeg.at[idx_d.at[io]], ssem).wait()
    plsc.subcore_barrier()
    pltpu.sync_copy(sdeg.at[slab], deg_out.at[pl.ds(c * NPAD + s * SLAB, SLAB)])


def _make_scatter(ds, nr, ni, slag, gla, ila, tc_tiling=True):
    assert ni >= ila + slag + 1 and nr >= gla + slag

    @functools.partial(
        pl.kernel,
        out_type=jax.ShapeDtypeStruct((NC * NPAD, ds), jnp.float32),
        mesh=_MESH,
        compiler_params=pltpu.CompilerParams(use_tc_tiling_on_sc=tc_tiling),
        scratch_types=[
            pltpu.VMEM((ni, CH), jnp.int32),
            pltpu.VMEM((ni, CH), jnp.int32),
            pltpu.VMEM((nr, CH, ds), jnp.float32),
            pltpu.SemaphoreType.DMA,
            pltpu.SemaphoreType.DMA,
            pltpu.SemaphoreType.DMA,
            pltpu.VMEM_SHARED((NPAD, ds), jnp.float32),
        ],
    )
    def _scatter(h, src1d, dst1d, acc_out,
                 idx_s, idx_d, rows, gsem, isem, ssem, sacc):
        c = lax.axis_index("c")
        s = lax.axis_index("s")
        t = c * NS + s
        slab = pl.ds(s * SLAB, SLAB)
        z16 = jnp.zeros((16,), dtype=jnp.float32)

        def fill(r, carry):
            for j in range(ds // 16):
                rows[0, r, pl.ds(j * 16, 16)] = z16
            return carry

        lax.fori_loop(0, CH, fill, 0)
        for k in range(SLAB // CH):
            pltpu.sync_copy(rows.at[0], sacc.at[pl.ds(s * SLAB + k * CH, CH)])
        plsc.subcore_barrier()

        def load_idx(j, slot):
            base = pl.multiple_of(t * ET + j * CH, CH)
            pltpu.async_copy(src1d.at[pl.ds(base, CH)], idx_s.at[slot], isem)
            pltpu.async_copy(dst1d.at[pl.ds(base, CH)], idx_d.at[slot], isem)

        def wait_idx(slot):
            pltpu.make_async_copy(src1d.at[pl.ds(0, CH)], idx_s.at[slot], isem).wait()
            pltpu.make_async_copy(dst1d.at[pl.ds(0, CH)], idx_d.at[slot], isem).wait()

        def gather(slot_i, slot_r):
            pltpu.async_copy(h.at[idx_s.at[slot_i]], rows.at[slot_r], gsem)

        def wait_gather(slot_i, slot_r):
            pltpu.make_async_copy(
                h.at[idx_s.at[slot_i]], rows.at[slot_r], gsem
            ).wait()

        def scat(slot_i, slot_r):
            pltpu.async_copy(
                rows.at[slot_r], sacc.at[idx_d.at[slot_i]], ssem, add=True
            )

        def wait_scat(slot_i, slot_r):
            pltpu.make_async_copy(
                rows.at[slot_r], sacc.at[idx_d.at[slot_i]], ssem
            ).wait()

        for j in range(ila):
            load_idx(j, j)
        for j in range(gla):
            wait_idx(j)
            gather(j, j)

        def body(i, carry):
            ri = lax.rem(i, nr)
            ii = lax.rem(i, ni)
            wait_gather(ii, ri)
            scat(ii, ri)

            @pl.when(i - slag >= 0)
            def _():
                wait_scat(lax.rem(i - slag, ni), lax.rem(i - slag, nr))

            @pl.when(i + gla < NCH)
            def _():
                wait_idx(lax.rem(i + gla, ni))
                gather(lax.rem(i + gla, ni), lax.rem(i + gla, nr))

            @pl.when(i + ila < NCH)
            def _():
                load_idx(i + ila, lax.rem(i + ila, ni))

            return carry

        lax.fori_loop(0, NCH, body, 0)
        for j in range(slag, 0, -1):
            wait_scat(lax.rem(NCH - j, ni), lax.rem(NCH - j, nr))
        plsc.subcore_barrier()
        pltpu.sync_copy(
            sacc.at[slab], acc_out.at[pl.ds(c * NPAD + s * SLAB, SLAB)]
        )

    return _scatter


_scatter_l1 = _make_scatter(ds=DW, nr=4, ni=6, slag=2, gla=2, ila=3)
_scatter_l2 = _make_scatter(ds=D_OUT, nr=6, ni=10, slag=3, gla=3, ila=6,
                            tc_tiling=False)


def _k1_body(x_ref, deg2_ref, w_ref, o_ref, dinv_ref):
    d2 = deg2_ref[...]
    d = d2[:, 0:1] + d2[:, 1:2] + 1.0
    dinv = lax.rsqrt(d)
    dinv_ref[...] = dinv
    h = jnp.dot(x_ref[...], w_ref[...], preferred_element_type=jnp.float32)
    o_ref[...] = dinv * h


def _k2_body(acc_ref, h1p_ref, dinv_ref, b1_ref, w2_ref, o_ref):
    dinv = dinv_ref[...]
    t = dinv * (acc_ref[0] + acc_ref[1] + h1p_ref[...]) + b1_ref[...]
    r = jnp.maximum(t, 0.0)
    h2 = jnp.dot(r, w2_ref[...], preferred_element_type=jnp.float32)
    o_ref[...] = dinv * h2


def _k3_body(acc_ref, h2p_ref, dinv_ref, b2_ref, o_ref):
    acc = acc_ref[0] + acc_ref[1] + h2p_ref[...]
    t = dinv_ref[...] * acc + b2_ref[...]
    m = jnp.max(t, axis=1, keepdims=True)
    e = jnp.exp(t - m)
    ssum = jnp.sum(e, axis=1, keepdims=True)
    o_ref[...] = (t - m) - jnp.log(ssum)


def _full(shape):
    return pl.BlockSpec(shape, lambda i: tuple(0 for _ in shape))


def kernel(x, edge_index, W1, b1, W2, b2):
    ei = edge_index.astype(jnp.int32)
    src1d = ei[0]
    dst1d = ei[1]

    degflat = _deg_kernel(dst1d)
    deg2 = degflat.reshape(NC, NPAD).T[:N]

    h1p, dinv = pl.pallas_call(
        _k1_body,
        out_shape=[
            jax.ShapeDtypeStruct((N, D_HID), jnp.float32),
            jax.ShapeDtypeStruct((N, 1), jnp.float32),
        ],
    )(x, deg2, W1)

    acc1 = _scatter_l1(h1p, src1d, dst1d).reshape(NC, NPAD, DW)

    h2p = pl.pallas_call(
        _k2_body,
        grid=(1,),
        in_specs=[
            _full((NC, N, DW)),
            _full((N, D_HID)),
            _full((N, 1)),
            _full((1, D_HID)),
            _full((D_HID, D_OUT)),
        ],
        out_specs=_full((N, D_OUT)),
        out_shape=jax.ShapeDtypeStruct((N, D_OUT), jnp.float32),
    )(acc1, h1p, dinv, b1.reshape(1, D_HID), W2)

    acc2 = _scatter_l2(h2p, src1d, dst1d).reshape(NC, NPAD, D_OUT)

    out = pl.pallas_call(
        _k3_body,
        grid=(1,),
        in_specs=[
            _full((NC, N, D_OUT)),
            _full((N, D_OUT)),
            _full((N, 1)),
            _full((1, D_OUT)),
        ],
        out_specs=_full((N, D_OUT)),
        out_shape=jax.ShapeDtypeStruct((N, D_OUT), jnp.float32),
    )(acc2, h2p, dinv, b2.reshape(1, D_OUT))
    return out


# speedup vs baseline: 1.0502x; 1.0209x over previous
import functools

import jax
import jax.numpy as jnp
from jax import lax
from jax.experimental import pallas as pl
from jax.experimental.pallas import tpu as pltpu
from jax.experimental.pallas import tpu_sc as plsc

N = 10000
E = 320000
D_IN = 128
D_HID = 128
D_OUT = 64
DW = 128

NC = 2
NS = 16
NT = NC * NS
NPAD = 10240
SLAB = NPAD // NS
CH = 80
NCH = (E // NT) // CH

_MESH = plsc.VectorSubcoreMesh(
    core_axis_name="c", subcore_axis_name="s", num_cores=NC, num_subcores=NS
)


ET = E // NT
DNI = 8
DILA = 4
DSLAG = 3


@functools.partial(
    pl.kernel,
    out_type=jax.ShapeDtypeStruct((NC * NPAD,), jnp.float32),
    mesh=_MESH,
    scratch_types=[
        pltpu.VMEM((DNI, CH), jnp.int32),
        pltpu.VMEM((CH,), jnp.float32),
        pltpu.VMEM((SLAB,), jnp.float32),
        pltpu.SemaphoreType.DMA,
        pltpu.SemaphoreType.DMA,
        pltpu.VMEM_SHARED((NPAD,), jnp.float32),
    ],
)
def _deg_kernel(dst1d, deg_out, idx_d, ones_v, zbuf, isem, ssem, sdeg):
    c = lax.axis_index("c")
    s = lax.axis_index("s")
    t = c * NS + s
    slab = pl.ds(s * SLAB, SLAB)
    z16 = jnp.zeros((16,), dtype=jnp.float32)
    ones16 = jnp.full((16,), 1.0, dtype=jnp.float32)

    def fill(i, carry):
        zbuf[pl.ds(i * 16, 16)] = z16
        return carry

    lax.fori_loop(0, SLAB // 16, fill, 0)
    for j in range(CH // 16):
        ones_v[pl.ds(j * 16, 16)] = ones16
    pltpu.sync_copy(zbuf, sdeg.at[slab])

    def load_idx(j, slot):
        base = pl.multiple_of(t * ET + j * CH, CH)
        pltpu.async_copy(dst1d.at[pl.ds(base, CH)], idx_d.at[slot], isem)

    def wait_idx(slot):
        pltpu.make_async_copy(dst1d.at[pl.ds(0, CH)], idx_d.at[slot], isem).wait()

    for j in range(DILA):
        load_idx(j, j)
    plsc.subcore_barrier()

    def body(i, carry):
        ii = lax.rem(i, DNI)
        wait_idx(ii)
        pltpu.sync_copy(ones_v, sdeg.at[idx_d.at[ii]], add=True)

        @pl.when(i + DILA < NCH)
        def _():
            load_idx(i + DILA, lax.rem(i + DILA, DNI))

        return carry

    lax.fori_loop(0, NCH, body, 0)
    plsc.subcore_barrier()
    pltpu.sync_copy(sdeg.at[slab], deg_out.at[pl.ds(c * NPAD + s * SLAB, SLAB)])


def _make_scatter(ds, nr, ni, slag, gla, ila, tc_tiling=True):
    assert ni >= ila + slag + 1 and nr >= gla + slag

    @functools.partial(
        pl.kernel,
        out_type=jax.ShapeDtypeStruct((NC * NPAD, ds), jnp.float32),
        mesh=_MESH,
        compiler_params=pltpu.CompilerParams(use_tc_tiling_on_sc=tc_tiling),
        scratch_types=[
            pltpu.VMEM((ni, CH), jnp.int32),
            pltpu.VMEM((ni, CH), jnp.int32),
            pltpu.VMEM((nr, CH, ds), jnp.float32),
            pltpu.SemaphoreType.DMA,
            pltpu.SemaphoreType.DMA,
            pltpu.SemaphoreType.DMA,
            pltpu.VMEM_SHARED((NPAD, ds), jnp.float32),
        ],
    )
    def _scatter(h, src1d, dst1d, acc_out,
                 idx_s, idx_d, rows, gsem, isem, ssem, sacc):
        c = lax.axis_index("c")
        s = lax.axis_index("s")
        t = c * NS + s
        slab = pl.ds(s * SLAB, SLAB)
        z16 = jnp.zeros((16,), dtype=jnp.float32)

        def fill(r, carry):
            for j in range(ds // 16):
                rows[0, r, pl.ds(j * 16, 16)] = z16
            return carry

        lax.fori_loop(0, CH, fill, 0)
        for k in range(SLAB // CH):
            pltpu.sync_copy(rows.at[0], sacc.at[pl.ds(s * SLAB + k * CH, CH)])
        plsc.subcore_barrier()

        def load_idx(j, slot):
            base = pl.multiple_of(t * ET + j * CH, CH)
            pltpu.async_copy(src1d.at[pl.ds(base, CH)], idx_s.at[slot], isem)
            pltpu.async_copy(dst1d.at[pl.ds(base, CH)], idx_d.at[slot], isem)

        def wait_idx(slot):
            pltpu.make_async_copy(src1d.at[pl.ds(0, CH)], idx_s.at[slot], isem).wait()
            pltpu.make_async_copy(dst1d.at[pl.ds(0, CH)], idx_d.at[slot], isem).wait()

        def gather(slot_i, slot_r):
            pltpu.async_copy(h.at[idx_s.at[slot_i]], rows.at[slot_r], gsem)

        def wait_gather(slot_i, slot_r):
            pltpu.make_async_copy(
                h.at[idx_s.at[slot_i]], rows.at[slot_r], gsem
            ).wait()

        def scat(slot_i, slot_r):
            pltpu.async_copy(
                rows.at[slot_r], sacc.at[idx_d.at[slot_i]], ssem, add=True
            )

        def wait_scat(slot_i, slot_r):
            pltpu.make_async_copy(
                rows.at[slot_r], sacc.at[idx_d.at[slot_i]], ssem
            ).wait()

        for j in range(ila):
            load_idx(j, j)
        for j in range(gla):
            wait_idx(j)
            gather(j, j)

        def body(i, carry):
            ri = lax.rem(i, nr)
            ii = lax.rem(i, ni)
            wait_gather(ii, ri)
            scat(ii, ri)

            @pl.when(i - slag >= 0)
            def _():
                wait_scat(lax.rem(i - slag, ni), lax.rem(i - slag, nr))

            @pl.when(i + gla < NCH)
            def _():
                wait_idx(lax.rem(i + gla, ni))
                gather(lax.rem(i + gla, ni), lax.rem(i + gla, nr))

            @pl.when(i + ila < NCH)
            def _():
                load_idx(i + ila, lax.rem(i + ila, ni))

            return carry

        lax.fori_loop(0, NCH, body, 0)
        for j in range(slag, 0, -1):
            wait_scat(lax.rem(NCH - j, ni), lax.rem(NCH - j, nr))
        plsc.subcore_barrier()
        pltpu.sync_copy(
            sacc.at[slab], acc_out.at[pl.ds(c * NPAD + s * SLAB, SLAB)]
        )

    return _scatter


_scatter_l1 = _make_scatter(ds=DW, nr=4, ni=6, slag=2, gla=2, ila=3)
_scatter_l2 = _make_scatter(ds=D_OUT, nr=6, ni=10, slag=3, gla=3, ila=6,
                            tc_tiling=False)


def _k1_body(x_ref, deg2_ref, w_ref, o_ref, dinv_ref):
    d2 = deg2_ref[...]
    d = d2[:, 0:1] + d2[:, 1:2] + 1.0
    dinv = lax.rsqrt(d)
    dinv_ref[...] = dinv
    h = jnp.dot(x_ref[...], w_ref[...], preferred_element_type=jnp.float32)
    o_ref[...] = dinv * h


def _k2_body(acc_ref, h1p_ref, dinv_ref, b1_ref, w2_ref, o_ref):
    dinv = dinv_ref[...]
    t = dinv * (acc_ref[0] + acc_ref[1] + h1p_ref[...]) + b1_ref[...]
    r = jnp.maximum(t, 0.0)
    h2 = jnp.dot(r, w2_ref[...], preferred_element_type=jnp.float32)
    o_ref[...] = dinv * h2


def _k3_body(acc_ref, h2p_ref, dinv2_ref, b2_ref, o_ref):
    acc = acc_ref[0] + acc_ref[1] + h2p_ref[...]
    d2 = dinv2_ref[...]
    b2 = b2_ref[...]

    def half(x, d):
        t = d * x + b2
        m = jnp.max(t, axis=1, keepdims=True)
        e = jnp.exp(t - m)
        ssum = jnp.sum(e, axis=1, keepdims=True)
        return (t - m) - jnp.log(ssum)

    o_ref[...] = jnp.concatenate(
        [half(acc[:, :D_OUT], d2[:, 0:1]), half(acc[:, D_OUT:], d2[:, 1:2])],
        axis=1,
    )


def _full(shape):
    return pl.BlockSpec(shape, lambda i: tuple(0 for _ in shape))


def kernel(x, edge_index, W1, b1, W2, b2):
    ei = edge_index.astype(jnp.int32)
    src1d = ei[0]
    dst1d = ei[1]

    degflat = _deg_kernel(dst1d)
    deg2 = degflat.reshape(NC, NPAD).T[:N]

    h1p, dinv = pl.pallas_call(
        _k1_body,
        out_shape=[
            jax.ShapeDtypeStruct((N, D_HID), jnp.float32),
            jax.ShapeDtypeStruct((N, 1), jnp.float32),
        ],
    )(x, deg2, W1)

    acc1 = _scatter_l1(h1p, src1d, dst1d).reshape(NC, NPAD, DW)

    h2p = pl.pallas_call(
        _k2_body,
        grid=(1,),
        in_specs=[
            _full((NC, N, DW)),
            _full((N, D_HID)),
            _full((N, 1)),
            _full((1, D_HID)),
            _full((D_HID, D_OUT)),
        ],
        out_specs=_full((N, D_OUT)),
        out_shape=jax.ShapeDtypeStruct((N, D_OUT), jnp.float32),
    )(acc1, h1p, dinv, b1.reshape(1, D_HID), W2)

    acc2 = _scatter_l2(h2p, src1d, dst1d).reshape(NC, NPAD // 2, DW)
    h2pv = h2p.reshape(N // 2, DW)
    dinv2 = dinv.reshape(N // 2, 2)

    out = pl.pallas_call(
        _k3_body,
        grid=(1,),
        in_specs=[
            _full((NC, N // 2, DW)),
            _full((N // 2, DW)),
            _full((N // 2, 2)),
            _full((1, D_OUT)),
        ],
        out_specs=_full((N // 2, DW)),
        out_shape=jax.ShapeDtypeStruct((N // 2, DW), jnp.float32),
    )(acc2, h2pv, dinv2, b2.reshape(1, D_OUT))
    return out.reshape(N, D_OUT)


# speedup vs baseline: 1.0789x; 1.0273x over previous
import functools

import jax
import jax.numpy as jnp
from jax import lax
from jax.experimental import pallas as pl
from jax.experimental.pallas import tpu as pltpu
from jax.experimental.pallas import tpu_sc as plsc

N = 10000
E = 320000
D_IN = 128
D_HID = 128
D_OUT = 64
DW = 128

NC = 2
NS = 16
NT = NC * NS
NPAD = 10240
SLAB = NPAD // NS
CH = 80
NCH = (E // NT) // CH

_MESH = plsc.VectorSubcoreMesh(
    core_axis_name="c", subcore_axis_name="s", num_cores=NC, num_subcores=NS
)


ET = E // NT
DNI = 8
DILA = 4
DSLAG = 3


@functools.partial(
    pl.kernel,
    out_type=jax.ShapeDtypeStruct((NC * NPAD,), jnp.float32),
    mesh=_MESH,
    scratch_types=[
        pltpu.VMEM((NCH, CH), jnp.int32),
        pltpu.VMEM((CH,), jnp.float32),
        pltpu.VMEM((SLAB,), jnp.float32),
        pltpu.VMEM_SHARED((NPAD,), jnp.float32),
    ],
)
def _deg_kernel(dst3d, deg_out, idx_d, ones_v, zbuf, sdeg):
    c = lax.axis_index("c")
    s = lax.axis_index("s")
    t = c * NS + s
    slab = pl.ds(s * SLAB, SLAB)
    z16 = jnp.zeros((16,), dtype=jnp.float32)
    ones16 = jnp.full((16,), 1.0, dtype=jnp.float32)

    def fill(i, carry):
        zbuf[pl.ds(i * 16, 16)] = z16
        return carry

    lax.fori_loop(0, SLAB // 16, fill, 0)
    for j in range(CH // 16):
        ones_v[pl.ds(j * 16, 16)] = ones16
    pltpu.sync_copy(zbuf, sdeg.at[slab])
    pltpu.sync_copy(dst3d.at[t], idx_d)
    plsc.subcore_barrier()

    def body(i, carry):
        pltpu.sync_copy(ones_v, sdeg.at[idx_d.at[i]], add=True)
        return carry

    lax.fori_loop(0, NCH, body, 0)
    plsc.subcore_barrier()
    pltpu.sync_copy(sdeg.at[slab], deg_out.at[pl.ds(c * NPAD + s * SLAB, SLAB)])


def _make_scatter(ds, nr, ni, slag, gla, ila, tc_tiling=True):
    assert ni >= ila + slag + 1 and nr >= gla + slag

    @functools.partial(
        pl.kernel,
        out_type=jax.ShapeDtypeStruct((NC * NPAD, ds), jnp.float32),
        mesh=_MESH,
        compiler_params=pltpu.CompilerParams(use_tc_tiling_on_sc=tc_tiling),
        scratch_types=[
            pltpu.VMEM((ni, CH), jnp.int32),
            pltpu.VMEM((ni, CH), jnp.int32),
            pltpu.VMEM((nr, CH, ds), jnp.float32),
            pltpu.SemaphoreType.DMA,
            pltpu.SemaphoreType.DMA,
            pltpu.SemaphoreType.DMA,
            pltpu.VMEM_SHARED((NPAD, ds), jnp.float32),
        ],
    )
    def _scatter(h, src3d, dst3d, acc_out,
                 idx_s, idx_d, rows, gsem, isem, ssem, sacc):
        c = lax.axis_index("c")
        s = lax.axis_index("s")
        t = c * NS + s
        slab = pl.ds(s * SLAB, SLAB)
        z16 = jnp.zeros((16,), dtype=jnp.float32)

        def fill(r, carry):
            for j in range(ds // 16):
                rows[0, r, pl.ds(j * 16, 16)] = z16
            return carry

        lax.fori_loop(0, CH, fill, 0)
        for k in range(SLAB // CH):
            pltpu.sync_copy(rows.at[0], sacc.at[pl.ds(s * SLAB + k * CH, CH)])
        plsc.subcore_barrier()

        def load_idx(j, slot):
            pltpu.async_copy(src3d.at[t, j], idx_s.at[slot], isem)
            pltpu.async_copy(dst3d.at[t, j], idx_d.at[slot], isem)

        def wait_idx(slot):
            pltpu.make_async_copy(src3d.at[t, 0], idx_s.at[slot], isem).wait()
            pltpu.make_async_copy(dst3d.at[t, 0], idx_d.at[slot], isem).wait()

        def gather(slot_i, slot_r):
            pltpu.async_copy(h.at[idx_s.at[slot_i]], rows.at[slot_r], gsem)

        def wait_gather(slot_i, slot_r):
            pltpu.make_async_copy(
                h.at[idx_s.at[slot_i]], rows.at[slot_r], gsem
            ).wait()

        def scat(slot_i, slot_r):
            pltpu.async_copy(
                rows.at[slot_r], sacc.at[idx_d.at[slot_i]], ssem, add=True
            )

        def wait_scat(slot_i, slot_r):
            pltpu.make_async_copy(
                rows.at[slot_r], sacc.at[idx_d.at[slot_i]], ssem
            ).wait()

        for j in range(ila):
            load_idx(j, j)
        for j in range(gla):
            wait_idx(j)
            gather(j, j)

        def body(i, carry):
            ri = lax.rem(i, nr)
            ii = lax.rem(i, ni)
            wait_gather(ii, ri)
            scat(ii, ri)

            @pl.when(i - slag >= 0)
            def _():
                wait_scat(lax.rem(i - slag, ni), lax.rem(i - slag, nr))

            @pl.when(i + gla < NCH)
            def _():
                wait_idx(lax.rem(i + gla, ni))
                gather(lax.rem(i + gla, ni), lax.rem(i + gla, nr))

            @pl.when(i + ila < NCH)
            def _():
                load_idx(i + ila, lax.rem(i + ila, ni))

            return carry

        lax.fori_loop(0, NCH, body, 0)
        for j in range(slag, 0, -1):
            wait_scat(lax.rem(NCH - j, ni), lax.rem(NCH - j, nr))
        plsc.subcore_barrier()
        pltpu.sync_copy(
            sacc.at[slab], acc_out.at[pl.ds(c * NPAD + s * SLAB, SLAB)]
        )

    return _scatter


_scatter_l1 = _make_scatter(ds=DW, nr=4, ni=6, slag=2, gla=2, ila=3)
_scatter_l2 = _make_scatter(ds=D_OUT, nr=6, ni=10, slag=3, gla=3, ila=6,
                            tc_tiling=False)


def _k1_body(x_ref, deg2_ref, w_ref, o_ref, dinv_ref):
    d2 = deg2_ref[...]
    d = d2[:, 0:1] + d2[:, 1:2] + 1.0
    dinv = lax.rsqrt(d)
    dinv_ref[...] = dinv
    h = jnp.dot(x_ref[...], w_ref[...], preferred_element_type=jnp.float32)
    o_ref[...] = dinv * h


def _k2_body(acc_ref, h1p_ref, dinv_ref, b1_ref, w2_ref, o_ref):
    dinv = dinv_ref[...]
    t = dinv * (acc_ref[0] + acc_ref[1] + h1p_ref[...]) + b1_ref[...]
    r = jnp.maximum(t, 0.0)
    h2 = jnp.dot(r, w2_ref[...], preferred_element_type=jnp.float32)
    o_ref[...] = dinv * h2


def _k3_body(acc_ref, h2p_ref, dinv2_ref, b2_ref, o_ref):
    acc = acc_ref[0] + acc_ref[1] + h2p_ref[...]
    d2 = dinv2_ref[...]
    b2 = b2_ref[...]

    def half(x, d):
        t = d * x + b2
        m = jnp.max(t, axis=1, keepdims=True)
        e = jnp.exp(t - m)
        ssum = jnp.sum(e, axis=1, keepdims=True)
        return (t - m) - jnp.log(ssum)

    o_ref[...] = jnp.concatenate(
        [half(acc[:, :D_OUT], d2[:, 0:1]), half(acc[:, D_OUT:], d2[:, 1:2])],
        axis=1,
    )


def _full(shape):
    return pl.BlockSpec(shape, lambda i: tuple(0 for _ in shape))


def kernel(x, edge_index, W1, b1, W2, b2):
    ei = edge_index.astype(jnp.int32)
    src3d = ei[0].reshape(NT, NCH, CH)
    dst3d = ei[1].reshape(NT, NCH, CH)

    degflat = _deg_kernel(dst3d)
    deg2 = degflat.reshape(NC, NPAD).T[:N]

    h1p, dinv = pl.pallas_call(
        _k1_body,
        out_shape=[
            jax.ShapeDtypeStruct((N, D_HID), jnp.float32),
            jax.ShapeDtypeStruct((N, 1), jnp.float32),
        ],
    )(x, deg2, W1)

    acc1 = _scatter_l1(h1p, src3d, dst3d).reshape(NC, NPAD, DW)

    h2p = pl.pallas_call(
        _k2_body,
        grid=(1,),
        in_specs=[
            _full((NC, N, DW)),
            _full((N, D_HID)),
            _full((N, 1)),
            _full((1, D_HID)),
            _full((D_HID, D_OUT)),
        ],
        out_specs=_full((N, D_OUT)),
        out_shape=jax.ShapeDtypeStruct((N, D_OUT), jnp.float32),
    )(acc1, h1p, dinv, b1.reshape(1, D_HID), W2)

    acc2 = _scatter_l2(h2p, src3d, dst3d).reshape(NC, NPAD // 2, DW)
    h2pv = h2p.reshape(N // 2, DW)
    dinv2 = dinv.reshape(N // 2, 2)

    out = pl.pallas_call(
        _k3_body,
        grid=(1,),
        in_specs=[
            _full((NC, N // 2, DW)),
            _full((N // 2, DW)),
            _full((N // 2, 2)),
            _full((1, D_OUT)),
        ],
        out_specs=_full((N // 2, DW)),
        out_shape=jax.ShapeDtypeStruct((N // 2, DW), jnp.float32),
    )(acc2, h2pv, dinv2, b2.reshape(1, D_OUT))
    return out.reshape(N, D_OUT)


# speedup vs baseline: 1.0869x; 1.0074x over previous
import functools

import jax
import jax.numpy as jnp
from jax import lax
from jax.experimental import pallas as pl
from jax.experimental.pallas import tpu as pltpu
from jax.experimental.pallas import tpu_sc as plsc

N = 10000
E = 320000
D_IN = 128
D_HID = 128
D_OUT = 64
DW = 128

NC = 2
NS = 16
NT = NC * NS
NPAD = 10240
SLAB = NPAD // NS
CH = 80
NCH = (E // NT) // CH

_MESH = plsc.VectorSubcoreMesh(
    core_axis_name="c", subcore_axis_name="s", num_cores=NC, num_subcores=NS
)


ET = E // NT
DNI = 8
DILA = 4
DSLAG = 3


@functools.partial(
    pl.kernel,
    out_type=jax.ShapeDtypeStruct((NC * NPAD,), jnp.float32),
    mesh=_MESH,
    scratch_types=[
        pltpu.VMEM((NCH, CH), jnp.int32),
        pltpu.VMEM((CH,), jnp.float32),
        pltpu.VMEM((SLAB,), jnp.float32),
        pltpu.VMEM_SHARED((NPAD,), jnp.float32),
    ],
)
def _deg_kernel(dst3d, deg_out, idx_d, ones_v, zbuf, sdeg):
    c = lax.axis_index("c")
    s = lax.axis_index("s")
    t = c * NS + s
    slab = pl.ds(s * SLAB, SLAB)
    z16 = jnp.zeros((16,), dtype=jnp.float32)
    ones16 = jnp.full((16,), 1.0, dtype=jnp.float32)

    def fill(i, carry):
        zbuf[pl.ds(i * 16, 16)] = z16
        return carry

    lax.fori_loop(0, SLAB // 16, fill, 0)
    for j in range(CH // 16):
        ones_v[pl.ds(j * 16, 16)] = ones16
    pltpu.sync_copy(zbuf, sdeg.at[slab])
    pltpu.sync_copy(dst3d.at[t], idx_d)
    plsc.subcore_barrier()

    def body(i, carry):
        pltpu.sync_copy(ones_v, sdeg.at[idx_d.at[i]], add=True)
        return carry

    lax.fori_loop(0, NCH, body, 0)
    plsc.subcore_barrier()
    pltpu.sync_copy(sdeg.at[slab], deg_out.at[pl.ds(c * NPAD + s * SLAB, SLAB)])


def _make_scatter(ds, nr, ni, slag, gla, ila, tc_tiling=True):
    assert ni >= ila + slag + 1 and nr >= gla + slag

    @functools.partial(
        pl.kernel,
        out_type=jax.ShapeDtypeStruct((NC * NPAD, ds), jnp.float32),
        mesh=_MESH,
        compiler_params=pltpu.CompilerParams(use_tc_tiling_on_sc=tc_tiling),
        scratch_types=[
            pltpu.VMEM((ni, CH), jnp.int32),
            pltpu.VMEM((ni, CH), jnp.int32),
            pltpu.VMEM((nr, CH, ds), jnp.float32),
            pltpu.SemaphoreType.DMA,
            pltpu.SemaphoreType.DMA,
            pltpu.SemaphoreType.DMA,
            pltpu.VMEM_SHARED((NPAD, ds), jnp.float32),
        ],
    )
    def _scatter(h, src3d, dst3d, acc_out,
                 idx_s, idx_d, rows, gsem, isem, ssem, sacc):
        c = lax.axis_index("c")
        s = lax.axis_index("s")
        t = c * NS + s
        slab = pl.ds(s * SLAB, SLAB)
        z16 = jnp.zeros((16,), dtype=jnp.float32)

        def fill(r, carry):
            for j in range(ds // 16):
                rows[0, r, pl.ds(j * 16, 16)] = z16
            return carry

        lax.fori_loop(0, CH, fill, 0)
        for k in range(SLAB // CH):
            pltpu.sync_copy(rows.at[0], sacc.at[pl.ds(s * SLAB + k * CH, CH)])
        plsc.subcore_barrier()

        def load_idx(j, slot):
            pltpu.async_copy(src3d.at[t, j], idx_s.at[slot], isem)
            pltpu.async_copy(dst3d.at[t, j], idx_d.at[slot], isem)

        def wait_idx(slot):
            pltpu.make_async_copy(src3d.at[t, 0], idx_s.at[slot], isem).wait()
            pltpu.make_async_copy(dst3d.at[t, 0], idx_d.at[slot], isem).wait()

        def gather(slot_i, slot_r):
            pltpu.async_copy(h.at[idx_s.at[slot_i]], rows.at[slot_r], gsem)

        def wait_gather(slot_i, slot_r):
            pltpu.make_async_copy(
                h.at[idx_s.at[slot_i]], rows.at[slot_r], gsem
            ).wait()

        def scat(slot_i, slot_r):
            pltpu.async_copy(
                rows.at[slot_r], sacc.at[idx_d.at[slot_i]], ssem, add=True
            )

        def wait_scat(slot_i, slot_r):
            pltpu.make_async_copy(
                rows.at[slot_r], sacc.at[idx_d.at[slot_i]], ssem
            ).wait()

        for j in range(ila):
            load_idx(j, j)
        for j in range(gla):
            wait_idx(j)
            gather(j, j)

        def body(i, carry):
            ri = lax.rem(i, nr)
            ii = lax.rem(i, ni)
            wait_gather(ii, ri)
            scat(ii, ri)

            @pl.when(i - slag >= 0)
            def _():
                wait_scat(lax.rem(i - slag, ni), lax.rem(i - slag, nr))

            @pl.when(i + gla < NCH)
            def _():
                wait_idx(lax.rem(i + gla, ni))
                gather(lax.rem(i + gla, ni), lax.rem(i + gla, nr))

            @pl.when(i + ila < NCH)
            def _():
                load_idx(i + ila, lax.rem(i + ila, ni))

            return carry

        lax.fori_loop(0, NCH, body, 0)
        for j in range(slag, 0, -1):
            wait_scat(lax.rem(NCH - j, ni), lax.rem(NCH - j, nr))
        plsc.subcore_barrier()
        pltpu.sync_copy(
            sacc.at[slab], acc_out.at[pl.ds(c * NPAD + s * SLAB, SLAB)]
        )

    return _scatter


_scatter_l1 = _make_scatter(ds=DW, nr=4, ni=6, slag=2, gla=2, ila=3,
                            tc_tiling=False)
_scatter_l2 = _make_scatter(ds=D_OUT, nr=6, ni=10, slag=3, gla=3, ila=6,
                            tc_tiling=False)


def _k1_body(x_ref, deg2_ref, w_ref, o_ref, dinv_ref):
    d2 = deg2_ref[...]
    d = d2[:, 0:1] + d2[:, 1:2] + 1.0
    dinv = lax.rsqrt(d)
    dinv_ref[...] = dinv
    h = jnp.dot(x_ref[...], w_ref[...], preferred_element_type=jnp.float32)
    o_ref[...] = dinv * h


def _k2_body(acc_ref, h1p_ref, dinv_ref, b1_ref, w2_ref, o_ref):
    dinv = dinv_ref[...]
    t = dinv * (acc_ref[0] + acc_ref[1] + h1p_ref[...]) + b1_ref[...]
    r = jnp.maximum(t, 0.0)
    h2 = jnp.dot(r, w2_ref[...], preferred_element_type=jnp.float32)
    o_ref[...] = dinv * h2


def _k3_body(acc_ref, h2p_ref, dinv2_ref, b2_ref, o_ref):
    acc = acc_ref[0] + acc_ref[1] + h2p_ref[...]
    d2 = dinv2_ref[...]
    b2 = b2_ref[...]

    def half(x, d):
        t = d * x + b2
        m = jnp.max(t, axis=1, keepdims=True)
        e = jnp.exp(t - m)
        ssum = jnp.sum(e, axis=1, keepdims=True)
        return (t - m) - jnp.log(ssum)

    o_ref[...] = jnp.concatenate(
        [half(acc[:, :D_OUT], d2[:, 0:1]), half(acc[:, D_OUT:], d2[:, 1:2])],
        axis=1,
    )


def _full(shape):
    return pl.BlockSpec(shape, lambda i: tuple(0 for _ in shape))


def kernel(x, edge_index, W1, b1, W2, b2):
    ei = edge_index.astype(jnp.int32)
    src3d = ei[0].reshape(NT, NCH, CH)
    dst3d = ei[1].reshape(NT, NCH, CH)

    degflat = _deg_kernel(dst3d)
    deg2 = degflat.reshape(NC, NPAD).T[:N]

    h1p, dinv = pl.pallas_call(
        _k1_body,
        out_shape=[
            jax.ShapeDtypeStruct((N, D_HID), jnp.float32),
            jax.ShapeDtypeStruct((N, 1), jnp.float32),
        ],
    )(x, deg2, W1)

    acc1 = _scatter_l1(h1p, src3d, dst3d).reshape(NC, NPAD, DW)

    h2p = pl.pallas_call(
        _k2_body,
        grid=(1,),
        in_specs=[
            _full((NC, N, DW)),
            _full((N, D_HID)),
            _full((N, 1)),
            _full((1, D_HID)),
            _full((D_HID, D_OUT)),
        ],
        out_specs=_full((N, D_OUT)),
        out_shape=jax.ShapeDtypeStruct((N, D_OUT), jnp.float32),
    )(acc1, h1p, dinv, b1.reshape(1, D_HID), W2)

    acc2 = _scatter_l2(h2p, src3d, dst3d).reshape(NC, NPAD // 2, DW)
    h2pv = h2p.reshape(N // 2, DW)
    dinv2 = dinv.reshape(N // 2, 2)

    out = pl.pallas_call(
        _k3_body,
        grid=(1,),
        in_specs=[
            _full((NC, N // 2, DW)),
            _full((N // 2, DW)),
            _full((N // 2, 2)),
            _full((1, D_OUT)),
        ],
        out_specs=_full((N // 2, DW)),
        out_shape=jax.ShapeDtypeStruct((N // 2, DW), jnp.float32),
    )(acc2, h2pv, dinv2, b2.reshape(1, D_OUT))
    return out.reshape(N, D_OUT)


# speedup vs baseline: 1.0917x; 1.0044x over previous
import functools

import jax
import jax.numpy as jnp
from jax import lax
from jax.experimental import pallas as pl
from jax.experimental.pallas import tpu as pltpu
from jax.experimental.pallas import tpu_sc as plsc

N = 10000
E = 320000
D_IN = 128
D_HID = 128
D_OUT = 64
DW = 128

NC = 2
NS = 16
NT = NC * NS
NPAD = 10240
SLAB = NPAD // NS
CH = 80
NCH = (E // NT) // CH

_MESH = plsc.VectorSubcoreMesh(
    core_axis_name="c", subcore_axis_name="s", num_cores=NC, num_subcores=NS
)


ET = E // NT
DNI = 8
DILA = 4
DSLAG = 3


@functools.partial(
    pl.kernel,
    out_type=jax.ShapeDtypeStruct((NC * NPAD,), jnp.float32),
    mesh=_MESH,
    compiler_params=pltpu.CompilerParams(use_tc_tiling_on_sc=False),
    scratch_types=[
        pltpu.VMEM((NCH, CH), jnp.int32),
        pltpu.VMEM((CH,), jnp.float32),
        pltpu.VMEM((SLAB,), jnp.float32),
        pltpu.VMEM_SHARED((NPAD,), jnp.float32),
    ],
)
def _deg_kernel(dst3d, deg_out, idx_d, ones_v, zbuf, sdeg):
    c = lax.axis_index("c")
    s = lax.axis_index("s")
    t = c * NS + s
    slab = pl.ds(s * SLAB, SLAB)
    z16 = jnp.zeros((16,), dtype=jnp.float32)
    ones16 = jnp.full((16,), 1.0, dtype=jnp.float32)

    def fill(i, carry):
        zbuf[pl.ds(i * 16, 16)] = z16
        return carry

    lax.fori_loop(0, SLAB // 16, fill, 0)
    for j in range(CH // 16):
        ones_v[pl.ds(j * 16, 16)] = ones16
    pltpu.sync_copy(zbuf, sdeg.at[slab])
    pltpu.sync_copy(dst3d.at[t], idx_d)
    plsc.subcore_barrier()

    def body(i, carry):
        pltpu.sync_copy(ones_v, sdeg.at[idx_d.at[i]], add=True)
        return carry

    lax.fori_loop(0, NCH, body, 0)
    plsc.subcore_barrier()
    pltpu.sync_copy(sdeg.at[slab], deg_out.at[pl.ds(c * NPAD + s * SLAB, SLAB)])


def _make_scatter(ds, nr, ni, slag, gla, ila, tc_tiling=True):
    assert ni >= ila + slag + 1 and nr >= gla + slag

    @functools.partial(
        pl.kernel,
        out_type=jax.ShapeDtypeStruct((NC * NPAD, ds), jnp.float32),
        mesh=_MESH,
        compiler_params=pltpu.CompilerParams(use_tc_tiling_on_sc=tc_tiling),
        scratch_types=[
            pltpu.VMEM((ni, CH), jnp.int32),
            pltpu.VMEM((ni, CH), jnp.int32),
            pltpu.VMEM((nr, CH, ds), jnp.float32),
            pltpu.SemaphoreType.DMA,
            pltpu.SemaphoreType.DMA,
            pltpu.SemaphoreType.DMA,
            pltpu.VMEM_SHARED((NPAD, ds), jnp.float32),
        ],
    )
    def _scatter(h, src3d, dst3d, acc_out,
                 idx_s, idx_d, rows, gsem, isem, ssem, sacc):
        c = lax.axis_index("c")
        s = lax.axis_index("s")
        t = c * NS + s
        slab = pl.ds(s * SLAB, SLAB)
        z16 = jnp.zeros((16,), dtype=jnp.float32)

        def fill(r, carry):
            for j in range(ds // 16):
                rows[0, r, pl.ds(j * 16, 16)] = z16
            return carry

        lax.fori_loop(0, CH, fill, 0)
        for k in range(SLAB // CH):
            pltpu.sync_copy(rows.at[0], sacc.at[pl.ds(s * SLAB + k * CH, CH)])
        plsc.subcore_barrier()

        def load_idx(j, slot):
            pltpu.async_copy(src3d.at[t, j], idx_s.at[slot], isem)
            pltpu.async_copy(dst3d.at[t, j], idx_d.at[slot], isem)

        def wait_idx(slot):
            pltpu.make_async_copy(src3d.at[t, 0], idx_s.at[slot], isem).wait()
            pltpu.make_async_copy(dst3d.at[t, 0], idx_d.at[slot], isem).wait()

        def gather(slot_i, slot_r):
            pltpu.async_copy(h.at[idx_s.at[slot_i]], rows.at[slot_r], gsem)

        def wait_gather(slot_i, slot_r):
            pltpu.make_async_copy(
                h.at[idx_s.at[slot_i]], rows.at[slot_r], gsem
            ).wait()

        def scat(slot_i, slot_r):
            pltpu.async_copy(
                rows.at[slot_r], sacc.at[idx_d.at[slot_i]], ssem, add=True
            )

        def wait_scat(slot_i, slot_r):
            pltpu.make_async_copy(
                rows.at[slot_r], sacc.at[idx_d.at[slot_i]], ssem
            ).wait()

        for j in range(ila):
            load_idx(j, j)
        for j in range(gla):
            wait_idx(j)
            gather(j, j)

        def body(i, carry):
            ri = lax.rem(i, nr)
            ii = lax.rem(i, ni)
            wait_gather(ii, ri)
            scat(ii, ri)

            @pl.when(i - slag >= 0)
            def _():
                wait_scat(lax.rem(i - slag, ni), lax.rem(i - slag, nr))

            @pl.when(i + gla < NCH)
            def _():
                wait_idx(lax.rem(i + gla, ni))
                gather(lax.rem(i + gla, ni), lax.rem(i + gla, nr))

            @pl.when(i + ila < NCH)
            def _():
                load_idx(i + ila, lax.rem(i + ila, ni))

            return carry

        lax.fori_loop(0, NCH, body, 0)
        for j in range(slag, 0, -1):
            wait_scat(lax.rem(NCH - j, ni), lax.rem(NCH - j, nr))
        plsc.subcore_barrier()
        pltpu.sync_copy(
            sacc.at[slab], acc_out.at[pl.ds(c * NPAD + s * SLAB, SLAB)]
        )

    return _scatter


_scatter_l1 = _make_scatter(ds=DW, nr=4, ni=6, slag=2, gla=2, ila=3,
                            tc_tiling=False)
_scatter_l2 = _make_scatter(ds=D_OUT, nr=6, ni=10, slag=3, gla=3, ila=6,
                            tc_tiling=False)


def _k1_body(x_ref, deg2_ref, w_ref, o_ref, dinv_ref):
    d2 = deg2_ref[...]
    d = d2[:, 0:1] + d2[:, 1:2] + 1.0
    dinv = lax.rsqrt(d)
    dinv_ref[...] = dinv
    h = jnp.dot(x_ref[...], w_ref[...], preferred_element_type=jnp.float32)
    o_ref[...] = dinv * h


def _k2_body(acc_ref, h1p_ref, dinv_ref, b1_ref, w2_ref, o_ref):
    dinv = dinv_ref[...]
    t = dinv * (acc_ref[0] + acc_ref[1] + h1p_ref[...]) + b1_ref[...]
    r = jnp.maximum(t, 0.0)
    h2 = jnp.dot(r, w2_ref[...], preferred_element_type=jnp.float32)
    o_ref[...] = dinv * h2


def _k3_body(acc_ref, h2p_ref, dinv2_ref, b2_ref, o_ref):
    acc = acc_ref[0] + acc_ref[1] + h2p_ref[...]
    d2 = dinv2_ref[...]
    b2 = b2_ref[...]

    def half(x, d):
        t = d * x + b2
        m = jnp.max(t, axis=1, keepdims=True)
        e = jnp.exp(t - m)
        ssum = jnp.sum(e, axis=1, keepdims=True)
        return (t - m) - jnp.log(ssum)

    o_ref[...] = jnp.concatenate(
        [half(acc[:, :D_OUT], d2[:, 0:1]), half(acc[:, D_OUT:], d2[:, 1:2])],
        axis=1,
    )


def _full(shape):
    return pl.BlockSpec(shape, lambda i: tuple(0 for _ in shape))


def kernel(x, edge_index, W1, b1, W2, b2):
    ei = edge_index.astype(jnp.int32)
    src3d = ei[0].reshape(NT, NCH, CH)
    dst3d = ei[1].reshape(NT, NCH, CH)

    degflat = _deg_kernel(dst3d)
    deg2 = degflat.reshape(NC, NPAD).T[:N]

    h1p, dinv = pl.pallas_call(
        _k1_body,
        out_shape=[
            jax.ShapeDtypeStruct((N, D_HID), jnp.float32),
            jax.ShapeDtypeStruct((N, 1), jnp.float32),
        ],
    )(x, deg2, W1)

    acc1 = _scatter_l1(h1p, src3d, dst3d).reshape(NC, NPAD, DW)

    h2p = pl.pallas_call(
        _k2_body,
        grid=(1,),
        in_specs=[
            _full((NC, N, DW)),
            _full((N, D_HID)),
            _full((N, 1)),
            _full((1, D_HID)),
            _full((D_HID, D_OUT)),
        ],
        out_specs=_full((N, D_OUT)),
        out_shape=jax.ShapeDtypeStruct((N, D_OUT), jnp.float32),
    )(acc1, h1p, dinv, b1.reshape(1, D_HID), W2)

    acc2 = _scatter_l2(h2p, src3d, dst3d).reshape(NC, NPAD // 2, DW)
    h2pv = h2p.reshape(N // 2, DW)
    dinv2 = dinv.reshape(N // 2, 2)

    out = pl.pallas_call(
        _k3_body,
        grid=(1,),
        in_specs=[
            _full((NC, N // 2, DW)),
            _full((N // 2, DW)),
            _full((N // 2, 2)),
            _full((1, D_OUT)),
        ],
        out_specs=_full((N // 2, DW)),
        out_shape=jax.ShapeDtypeStruct((N // 2, DW), jnp.float32),
    )(acc2, h2pv, dinv2, b2.reshape(1, D_OUT))
    return out.reshape(N, D_OUT)


# speedup vs baseline: 1.0941x; 1.0022x over previous
import functools

import jax
import jax.numpy as jnp
from jax import lax
from jax.experimental import pallas as pl
from jax.experimental.pallas import tpu as pltpu
from jax.experimental.pallas import tpu_sc as plsc

N = 10000
E = 320000
D_IN = 128
D_HID = 128
D_OUT = 64
DW = 128

NC = 2
NS = 16
NT = NC * NS
NPAD = 10240
SLAB = NPAD // NS
CH = 80
NCH = (E // NT) // CH

_MESH = plsc.VectorSubcoreMesh(
    core_axis_name="c", subcore_axis_name="s", num_cores=NC, num_subcores=NS
)


ET = E // NT
DNI = 8
DILA = 4
DSLAG = 3


@functools.partial(
    pl.kernel,
    out_type=jax.ShapeDtypeStruct((NC * NPAD,), jnp.float32),
    mesh=_MESH,
    compiler_params=pltpu.CompilerParams(use_tc_tiling_on_sc=False),
    scratch_types=[
        pltpu.VMEM((NCH, CH), jnp.int32),
        pltpu.VMEM((CH,), jnp.float32),
        pltpu.VMEM((SLAB,), jnp.float32),
        pltpu.VMEM_SHARED((NPAD,), jnp.float32),
    ],
)
def _deg_kernel(dst3d, deg_out, idx_d, ones_v, zbuf, sdeg):
    c = lax.axis_index("c")
    s = lax.axis_index("s")
    t = c * NS + s
    slab = pl.ds(s * SLAB, SLAB)
    z16 = jnp.zeros((16,), dtype=jnp.float32)
    ones16 = jnp.full((16,), 1.0, dtype=jnp.float32)

    def fill(i, carry):
        zbuf[pl.ds(i * 16, 16)] = z16
        return carry

    lax.fori_loop(0, SLAB // 16, fill, 0)
    for j in range(CH // 16):
        ones_v[pl.ds(j * 16, 16)] = ones16
    pltpu.sync_copy(zbuf, sdeg.at[slab])
    pltpu.sync_copy(dst3d.at[t], idx_d)
    plsc.subcore_barrier()

    def body(i, carry):
        pltpu.sync_copy(ones_v, sdeg.at[idx_d.at[i]], add=True)
        return carry

    lax.fori_loop(0, NCH, body, 0)
    plsc.subcore_barrier()
    pltpu.sync_copy(sdeg.at[slab], deg_out.at[pl.ds(c * NPAD + s * SLAB, SLAB)])


def _make_scatter(ds, nr, ni, slag, gla, ila, tc_tiling=True):
    assert ni >= ila + slag + 1 and nr >= gla + slag

    @functools.partial(
        pl.kernel,
        out_type=jax.ShapeDtypeStruct((NC * NPAD, ds), jnp.float32),
        mesh=_MESH,
        compiler_params=pltpu.CompilerParams(use_tc_tiling_on_sc=tc_tiling),
        scratch_types=[
            pltpu.VMEM((ni, CH), jnp.int32),
            pltpu.VMEM((ni, CH), jnp.int32),
            pltpu.VMEM((nr, CH, ds), jnp.float32),
            pltpu.SemaphoreType.DMA,
            pltpu.SemaphoreType.DMA,
            pltpu.SemaphoreType.DMA,
            pltpu.VMEM_SHARED((NPAD, ds), jnp.float32),
        ],
    )
    def _scatter(h, src3d, dst3d, acc_out,
                 idx_s, idx_d, rows, gsem, isem, ssem, sacc):
        c = lax.axis_index("c")
        s = lax.axis_index("s")
        t = c * NS + s
        slab = pl.ds(s * SLAB, SLAB)
        z16 = jnp.zeros((16,), dtype=jnp.float32)

        def fill(r, carry):
            for j in range(ds // 16):
                rows[0, r, pl.ds(j * 16, 16)] = z16
            return carry

        lax.fori_loop(0, CH, fill, 0)
        for k in range(SLAB // CH):
            pltpu.sync_copy(rows.at[0], sacc.at[pl.ds(s * SLAB + k * CH, CH)])
        plsc.subcore_barrier()

        def load_idx(j, slot):
            pltpu.async_copy(src3d.at[t, j], idx_s.at[slot], isem)
            pltpu.async_copy(dst3d.at[t, j], idx_d.at[slot], isem)

        def wait_idx(slot):
            pltpu.make_async_copy(src3d.at[t, 0], idx_s.at[slot], isem).wait()
            pltpu.make_async_copy(dst3d.at[t, 0], idx_d.at[slot], isem).wait()

        def gather(slot_i, slot_r):
            pltpu.async_copy(h.at[idx_s.at[slot_i]], rows.at[slot_r], gsem)

        def wait_gather(slot_i, slot_r):
            pltpu.make_async_copy(
                h.at[idx_s.at[slot_i]], rows.at[slot_r], gsem
            ).wait()

        def scat(slot_i, slot_r):
            pltpu.async_copy(
                rows.at[slot_r], sacc.at[idx_d.at[slot_i]], ssem, add=True
            )

        def wait_scat(slot_i, slot_r):
            pltpu.make_async_copy(
                rows.at[slot_r], sacc.at[idx_d.at[slot_i]], ssem
            ).wait()

        for j in range(ila):
            load_idx(j, j)
        for j in range(gla):
            wait_idx(j)
            gather(j, j)

        def body(i, carry):
            ri = lax.rem(i, nr)
            ii = lax.rem(i, ni)
            wait_gather(ii, ri)
            scat(ii, ri)

            @pl.when(i - slag >= 0)
            def _():
                wait_scat(lax.rem(i - slag, ni), lax.rem(i - slag, nr))

            @pl.when(i + gla < NCH)
            def _():
                wait_idx(lax.rem(i + gla, ni))
                gather(lax.rem(i + gla, ni), lax.rem(i + gla, nr))

            @pl.when(i + ila < NCH)
            def _():
                load_idx(i + ila, lax.rem(i + ila, ni))

            return carry

        lax.fori_loop(0, NCH, body, 0)
        for j in range(slag, 0, -1):
            wait_scat(lax.rem(NCH - j, ni), lax.rem(NCH - j, nr))
        plsc.subcore_barrier()
        pltpu.sync_copy(
            sacc.at[slab], acc_out.at[pl.ds(c * NPAD + s * SLAB, SLAB)]
        )

    return _scatter


_scatter_l1 = _make_scatter(ds=DW, nr=4, ni=6, slag=2, gla=2, ila=3,
                            tc_tiling=False)
_scatter_l2 = _make_scatter(ds=D_OUT, nr=6, ni=10, slag=3, gla=3, ila=6,
                            tc_tiling=False)


def _k1_body(x_ref, deg2_ref, w_ref, o_ref, dinv_ref):
    d2 = deg2_ref[...]
    d = d2[:, 0:1] + d2[:, 1:2] + 1.0
    dinv = lax.rsqrt(d)
    dinv_ref[...] = dinv
    h = jnp.dot(x_ref[...], w_ref[...], preferred_element_type=jnp.float32)
    o_ref[...] = dinv * h


def _k2_body(acc_ref, h1p_ref, dinv_ref, b1_ref, w2_ref, o_ref):
    dinv = dinv_ref[...]
    t = dinv * (acc_ref[0] + acc_ref[1] + h1p_ref[...]) + b1_ref[...]
    r = jnp.maximum(t, 0.0)
    h2 = jnp.dot(r, w2_ref[...], preferred_element_type=jnp.float32)
    o_ref[...] = dinv * h2


def _k3_body(acc_ref, h2p_ref, dinv2_ref, b2_ref, o_ref):
    acc = acc_ref[0] + acc_ref[1] + h2p_ref[...]
    d2 = dinv2_ref[...]
    b2 = b2_ref[...]

    def half(x, d):
        t = d * x + b2
        m = jnp.max(t, axis=1, keepdims=True)
        e = jnp.exp(t - m)
        ssum = jnp.sum(e, axis=1, keepdims=True)
        return (t - m) - jnp.log(ssum)

    o_ref[...] = jnp.concatenate(
        [half(acc[:, :D_OUT], d2[:, 0:1]), half(acc[:, D_OUT:], d2[:, 1:2])],
        axis=1,
    )


def _full(shape):
    return pl.BlockSpec(shape, lambda i: tuple(0 for _ in shape))


def kernel(x, edge_index, W1, b1, W2, b2):
    ei = edge_index.astype(jnp.int32)
    src3d = lax.optimization_barrier(ei[0].reshape(NT, NCH, CH))
    dst3d = ei[1].reshape(NT, NCH, CH)

    degflat = _deg_kernel(dst3d)
    deg2 = degflat.reshape(NC, NPAD).T[:N]

    h1p, dinv = pl.pallas_call(
        _k1_body,
        out_shape=[
            jax.ShapeDtypeStruct((N, D_HID), jnp.float32),
            jax.ShapeDtypeStruct((N, 1), jnp.float32),
        ],
    )(x, deg2, W1)

    acc1 = _scatter_l1(h1p, src3d, dst3d).reshape(NC, NPAD, DW)

    h2p = pl.pallas_call(
        _k2_body,
        grid=(1,),
        in_specs=[
            _full((NC, N, DW)),
            _full((N, D_HID)),
            _full((N, 1)),
            _full((1, D_HID)),
            _full((D_HID, D_OUT)),
        ],
        out_specs=_full((N, D_OUT)),
        out_shape=jax.ShapeDtypeStruct((N, D_OUT), jnp.float32),
    )(acc1, h1p, dinv, b1.reshape(1, D_HID), W2)

    acc2 = _scatter_l2(h2p, src3d, dst3d).reshape(NC, NPAD // 2, DW)
    h2pv = h2p.reshape(N // 2, DW)
    dinv2 = dinv.reshape(N // 2, 2)

    out = pl.pallas_call(
        _k3_body,
        grid=(1,),
        in_specs=[
            _full((NC, N // 2, DW)),
            _full((N // 2, DW)),
            _full((N // 2, 2)),
            _full((1, D_OUT)),
        ],
        out_specs=_full((N // 2, DW)),
        out_shape=jax.ShapeDtypeStruct((N // 2, DW), jnp.float32),
    )(acc2, h2pv, dinv2, b2.reshape(1, D_OUT))
    return out.reshape(N, D_OUT)
